# baseline (speedup 1.0000x reference)
import functools

import jax
import jax.numpy as jnp
from jax import lax
from jax.experimental import pallas as pl
from jax.experimental.pallas import tpu as pltpu
from jax.experimental.pallas import tpu_sc as plsc

NUM_EMB = 1_000_000
EMBEDDING_DIM = 32
WIDE = 128
ROWS_PER_LINE = WIDE // EMBEDDING_DIM
NUM_LINES = NUM_EMB // ROWS_PER_LINE
NUM_RECORDS = NUM_LINES // 8
BATCH = 16384
NUM_CORES = 2
NUM_SUBCORES = 16
NUM_WORKERS = NUM_CORES * NUM_SUBCORES
B_PER_W = BATCH // NUM_WORKERS
CHUNK = 32
NCHUNK = B_PER_W // CHUNK
LANES = 16


@functools.partial(
    pl.kernel,
    mesh=plsc.VectorSubcoreMesh(core_axis_name="c", subcore_axis_name="s"),
    out_type=jax.ShapeDtypeStruct((BATCH, EMBEDDING_DIM), jnp.float32),
    scratch_types=[
        pltpu.VMEM((B_PER_W,), jnp.int32),
        pltpu.VMEM((B_PER_W,), jnp.int32),
        pltpu.VMEM((CHUNK, 8, WIDE), jnp.float32),
        pltpu.VMEM((CHUNK, 8, WIDE), jnp.float32),
        pltpu.VMEM((CHUNK, EMBEDDING_DIM), jnp.float32),
        pltpu.SemaphoreType.DMA,
        pltpu.SemaphoreType.DMA,
    ],
    compiler_params=pltpu.CompilerParams(
        use_tc_tiling_on_sc=True, needs_layout_passes=False
    ),
)
def _emb_lookup(tbl, idx_hbm, out, idx_v, rec_v, buf0, buf1, rows_v, sem0, sem1):
    wid = lax.axis_index("s") * NUM_CORES + lax.axis_index("c")
    base = wid * B_PER_W
    pltpu.sync_copy(idx_hbm.at[pl.ds(base, B_PER_W)], idx_v)
    for g in range(B_PER_W // LANES):
        sl = pl.ds(g * LANES, LANES)
        rec_v[sl] = lax.shift_right_logical(idx_v[sl], 5)
    t3 = tbl.reshape(NUM_RECORDS, 8, WIDE)
    bufs = (buf0, buf1)
    sems = (sem0, sem1)

    def fire(c, buf, sem):
        pltpu.async_copy(t3.at[rec_v.at[pl.ds(c * CHUNK, CHUNK)]], buf, sem)

    def drain(c, buf, sem):
        pltpu.make_async_copy(
            t3.at[rec_v.at[pl.ds(c * CHUNK, CHUNK)]], buf, sem
        ).wait()

    def extract(c, buf):
        for g in range(CHUNK // LANES):
            ivec = idx_v[pl.ds(c * CHUNK + g * LANES, LANES)]
            rvec = lax.bitwise_and(lax.shift_right_logical(ivec, 2), 7)
            cvec = lax.bitwise_and(ivec, 3) * EMBEDDING_DIM
            for kk in range(LANES):
                j = g * LANES + kk
                for dd in range(EMBEDDING_DIM // LANES):
                    cols = (
                        jnp.broadcast_to(cvec[kk], (LANES,))
                        + lax.iota(jnp.int32, LANES)
                        + dd * LANES
                    )
                    vals = plsc.load_gather(
                        buf,
                        [jnp.full((LANES,), j, jnp.int32),
                         jnp.broadcast_to(rvec[kk], (LANES,)),
                         cols],
                    )
                    rows_v[j, pl.ds(dd * LANES, LANES)] = vals

    fire(0, bufs[0], sems[0])
    for c in range(NCHUNK):
        if c + 1 < NCHUNK:
            fire(c + 1, bufs[(c + 1) % 2], sems[(c + 1) % 2])
        drain(c, bufs[c % 2], sems[c % 2])
        extract(c, bufs[c % 2])
        pltpu.sync_copy(rows_v, out.at[pl.ds(base + c * CHUNK, CHUNK)])


def kernel(x, weight):
    wide = weight.reshape(NUM_LINES, WIDE)
    return _emb_lookup(wide, x.astype(jnp.int32))


# speedup vs baseline: 3.6329x; 3.6329x over previous
import functools

import jax
import jax.numpy as jnp
from jax import lax
from jax.experimental import pallas as pl
from jax.experimental.pallas import tpu as pltpu
from jax.experimental.pallas import tpu_sc as plsc

NUM_EMB = 1_000_000
EMBEDDING_DIM = 32
BATCH = 16384
NUM_CORES = 2
NUM_SUBCORES = 16
NUM_WORKERS = NUM_CORES * NUM_SUBCORES
B_PER_W = BATCH // NUM_WORKERS
LANES = 16
CHUNK = 8
NPAIR = B_PER_W // (2 * CHUNK)
MAX_COL = (NUM_EMB // 128) * 128 - 128
TAIL_START = (NUM_EMB // 128) * 128


@functools.partial(
    pl.kernel,
    mesh=plsc.VectorSubcoreMesh(core_axis_name="c", subcore_axis_name="s"),
    out_type=jax.ShapeDtypeStruct((BATCH, EMBEDDING_DIM), jnp.float32),
    scratch_types=[
        pltpu.VMEM((B_PER_W,), jnp.int32),
        pltpu.VMEM((CHUNK, EMBEDDING_DIM, 128), jnp.float32),
        pltpu.VMEM((CHUNK, EMBEDDING_DIM, 128), jnp.float32),
        pltpu.VMEM((CHUNK, EMBEDDING_DIM), jnp.float32),
        pltpu.SemaphoreType.DMA,
        pltpu.SemaphoreType.DMA,
    ],
    compiler_params=pltpu.CompilerParams(
        use_tc_tiling_on_sc=True, needs_layout_passes=False
    ),
)
def _emb_lookup(wt, idx_hbm, out, idx_v, buf0, buf1, rows_v, sem0, sem1):
    wid = lax.axis_index("s") * NUM_CORES + lax.axis_index("c")
    base = wid * B_PER_W
    pltpu.sync_copy(idx_hbm.at[pl.ds(base, B_PER_W)], idx_v)
    bufs = (buf0, buf1)
    sems = (sem0, sem1)

    def pair_cols_lanes(p):
        ivec = idx_v[pl.ds(p * 2 * CHUNK, LANES)]
        cols = jnp.minimum(
            lax.shift_right_logical(ivec, 7) * 128,
            jnp.full((LANES,), MAX_COL, jnp.int32),
        )
        lanes_vec = jnp.minimum(ivec - cols, jnp.full((LANES,), 127, jnp.int32))
        return cols, lanes_vec

    def fire(cols, half, buf, sem):
        for kk in range(CHUNK):
            col = pl.multiple_of(cols[half * CHUNK + kk], 128)
            pltpu.async_copy(wt.at[:, pl.ds(col, 128)], buf.at[kk], sem)

    def drain(buf, sem):
        for kk in range(CHUNK):
            pltpu.make_async_copy(
                wt.at[:, pl.ds(0, 128)], buf.at[kk], sem
            ).wait()

    def extract(lanes_vec, half, buf):
        for kk in range(CHUNK):
            lane = lanes_vec[half * CHUNK + kk]
            for dd in range(EMBEDDING_DIM // LANES):
                rows = lax.iota(jnp.int32, LANES) + dd * LANES
                vals = plsc.load_gather(
                    buf,
                    [jnp.full((LANES,), kk, jnp.int32),
                     rows,
                     jnp.broadcast_to(lane, (LANES,))],
                )
                rows_v[kk, pl.ds(dd * LANES, LANES)] = vals

    cols0, _ = pair_cols_lanes(0)
    fire(cols0, 0, bufs[0], sems[0])
    fire(cols0, 1, bufs[1], sems[1])

    def body(p, carry):
        _, lanes_vec = pair_cols_lanes(p)
        ncols, _ = pair_cols_lanes(jnp.minimum(p + 1, NPAIR - 1))
        for half in range(2):
            drain(bufs[half], sems[half])
            extract(lanes_vec, half, bufs[half])
            c = p * 2 + half
            off = pl.multiple_of(base + c * CHUNK, 8)
            pltpu.sync_copy(rows_v, out.at[pl.ds(off, CHUNK)])

            @pl.when(p + 1 < NPAIR)
            def _():
                fire(ncols, half, bufs[half], sems[half])
        return carry

    lax.fori_loop(0, NPAIR, body, 0)


def kernel(x, weight):
    xi = x.astype(jnp.int32)
    main = _emb_lookup(weight.T, xi)
    tail_ids = jnp.clip(xi - TAIL_START, 0, NUM_EMB - TAIL_START - 1)
    tail = jnp.take(weight[TAIL_START:], tail_ids, axis=0)
    return jnp.where((xi >= TAIL_START)[:, None], tail, main)
